# baseline (speedup 1.0000x reference)
import functools

import jax
import jax.numpy as jnp
from jax import lax
from jax.experimental import pallas as pl
from jax.experimental.pallas import tpu as pltpu, tpu_sc as plsc

NUM_SETS = 32
SET_SIZE = 16
D = 128
NUM_GENES_USED = NUM_SETS * SET_SIZE
LANES = 16
DV = D // LANES

SC_BATCH = 96
TC_BLOCK = 8

HALF_SETS = NUM_SETS // 2
HALF_ROWS = NUM_GENES_USED // 2


def _softmax_body(a_ref, o_ref):
    x = a_ref[...]
    m = jnp.max(x, axis=1, keepdims=True)
    e = jnp.exp(x - m)
    o_ref[...] = e / jnp.sum(e, axis=1, keepdims=True)


def _tc_pool_body(x_ref, w_ref, o_ref):
    x = x_ref[...]
    w = w_ref[...]
    x4 = x.reshape(TC_BLOCK, NUM_SETS, SET_SIZE, D)
    o_ref[...] = jnp.sum(x4 * w[None, :, :, None], axis=2)


def _sc_body(gene_hbm, w_hbm, out_hbm, w_v, slab_a, slab_b, out_v, sem_a, sem_b):
    nc = 2
    wid = lax.axis_index("s") * nc + lax.axis_index("c")
    b = gene_hbm.shape[0]
    b_per_w = b // (nc * 16)
    base = wid * b_per_w

    pltpu.sync_copy(w_hbm, w_v)

    def start_half(bb, h, buf, sem):
        pltpu.make_async_copy(
            gene_hbm.at[bb, pl.ds(h * HALF_ROWS, HALF_ROWS), :], buf, sem
        ).start()

    def compute_half(buf, h):
        def set_body(sl, _):
            wvec = w_v[h * HALF_SETS + sl, :]
            accs = [jnp.zeros((LANES,), jnp.float32) for _ in range(DV)]
            for k in range(SET_SIZE):
                wk = wvec[k]
                row = sl * SET_SIZE + k
                for v in range(DV):
                    accs[v] = accs[v] + wk * buf[row, pl.ds(v * LANES, LANES)]
            for v in range(DV):
                out_v[h * HALF_SETS + sl, pl.ds(v * LANES, LANES)] = accs[v]
            return 0

        lax.fori_loop(0, HALF_SETS, set_body, 0)

    start_half(base, 0, slab_a, sem_a)

    def batch_body(i, _):
        bb = base + i
        start_half(bb, 1, slab_b, sem_b)
        pltpu.make_async_copy(
            gene_hbm.at[bb, pl.ds(0, HALF_ROWS), :], slab_a, sem_a
        ).wait()
        compute_half(slab_a, 0)

        @pl.when(i < b_per_w - 1)
        def _():
            start_half(bb + 1, 0, slab_a, sem_a)

        pltpu.make_async_copy(
            gene_hbm.at[bb, pl.ds(HALF_ROWS, HALF_ROWS), :], slab_b, sem_b
        ).wait()
        compute_half(slab_b, 1)
        pltpu.sync_copy(out_v, out_hbm.at[bb])
        return 0

    lax.fori_loop(0, b_per_w, batch_body, 0)


def _sc_pool(gene_sc, w):
    mesh = plsc.VectorSubcoreMesh(core_axis_name="c", subcore_axis_name="s")
    f = pl.kernel(
        _sc_body,
        out_type=jax.ShapeDtypeStruct((gene_sc.shape[0], NUM_SETS, D), jnp.float32),
        mesh=mesh,
        scratch_types=[
            pltpu.VMEM((NUM_SETS, SET_SIZE), jnp.float32),
            pltpu.VMEM((HALF_ROWS, D), jnp.float32),
            pltpu.VMEM((HALF_ROWS, D), jnp.float32),
            pltpu.VMEM((NUM_SETS, D), jnp.float32),
            pltpu.SemaphoreType.DMA,
            pltpu.SemaphoreType.DMA,
        ],
    )
    return f(gene_sc, w)


def _tc_pool(gene_features, w, n_tc):
    grid = (n_tc // TC_BLOCK,)
    return pl.pallas_call(
        _tc_pool_body,
        grid=grid,
        in_specs=[
            pl.BlockSpec((TC_BLOCK, NUM_GENES_USED, D), lambda i: (i, 0, 0)),
            pl.BlockSpec((NUM_SETS, SET_SIZE), lambda i: (0, 0)),
        ],
        out_specs=pl.BlockSpec((TC_BLOCK, NUM_SETS, D), lambda i: (i, 0, 0)),
        out_shape=jax.ShapeDtypeStruct((n_tc, NUM_SETS, D), jnp.float32),
    )(gene_features, w)


def kernel(gene_features, attn_w):
    b = gene_features.shape[0]
    n_tc = b - SC_BATCH
    attn2 = attn_w.reshape(NUM_SETS, SET_SIZE)
    w = pl.pallas_call(
        _softmax_body,
        out_shape=jax.ShapeDtypeStruct((NUM_SETS, SET_SIZE), jnp.float32),
    )(attn2)

    gene_sc = lax.slice(
        gene_features, (n_tc, 0, 0), (b, NUM_GENES_USED, D)
    )
    out_sc = _sc_pool(gene_sc, w)
    out_tc = _tc_pool(gene_features, w, n_tc)
    return jnp.concatenate([out_tc, out_sc], axis=0)


# speedup vs baseline: 2.7474x; 2.7474x over previous
import functools

import jax
import jax.numpy as jnp
from jax import lax
from jax.experimental import pallas as pl
from jax.experimental.pallas import tpu as pltpu, tpu_sc as plsc

NUM_SETS = 32
SET_SIZE = 16
D = 128
NUM_GENES_USED = NUM_SETS * SET_SIZE
LANES = 16
DV = D // LANES

BCHUNK = 16


def _softmax_body(a_ref, o_ref):
    x = a_ref[...]
    m = jnp.max(x, axis=1, keepdims=True)
    e = jnp.exp(x - m)
    o_ref[...] = e / jnp.sum(e, axis=1, keepdims=True)


def _sc_body(gene_hbm, w_hbm, out_hbm, w_v, slab_a, slab_b, out_v, sem_a, sem_b):
    nc = 2
    wid = lax.axis_index("s") * nc + lax.axis_index("c")
    b = gene_hbm.shape[1]
    n_chunks = b // BCHUNK
    g0 = wid * SET_SIZE

    pltpu.sync_copy(w_hbm, w_v)
    wvec = w_v[wid, :]
    g_src = gene_hbm.at[pl.ds(g0, SET_SIZE)]

    def start_chunk(c, buf, sem):
        pltpu.make_async_copy(
            g_src.at[:, pl.ds(c * BCHUNK, BCHUNK), :], buf, sem
        ).start()

    def compute_chunk(buf, c):
        def b_body(bl, _):
            accs = [jnp.zeros((LANES,), jnp.float32) for _ in range(DV)]
            for k in range(SET_SIZE):
                wk = wvec[k]
                for v in range(DV):
                    accs[v] = accs[v] + wk * buf[k, bl, pl.ds(v * LANES, LANES)]
            for v in range(DV):
                out_v[bl, pl.ds(v * LANES, LANES)] = accs[v]
            return 0

        lax.fori_loop(0, BCHUNK, b_body, 0)
        pltpu.sync_copy(out_v, out_hbm.at[wid, pl.ds(c * BCHUNK, BCHUNK), :])

    start_chunk(0, slab_a, sem_a)

    def pair_body(i, _):
        c0 = 2 * i
        start_chunk(c0 + 1, slab_b, sem_b)
        pltpu.make_async_copy(
            g_src.at[:, pl.ds(c0 * BCHUNK, BCHUNK), :], slab_a, sem_a
        ).wait()
        compute_chunk(slab_a, c0)

        @pl.when(c0 + 2 < n_chunks)
        def _():
            start_chunk(c0 + 2, slab_a, sem_a)

        pltpu.make_async_copy(
            g_src.at[:, pl.ds((c0 + 1) * BCHUNK, BCHUNK), :], slab_b, sem_b
        ).wait()
        compute_chunk(slab_b, c0 + 1)
        return 0

    lax.fori_loop(0, n_chunks // 2, pair_body, 0)


def kernel(gene_features, attn_w):
    b = gene_features.shape[0]
    attn2 = attn_w.reshape(NUM_SETS, SET_SIZE)
    w = pl.pallas_call(
        _softmax_body,
        out_shape=jax.ShapeDtypeStruct((NUM_SETS, SET_SIZE), jnp.float32),
    )(attn2)

    gf_t = jnp.transpose(gene_features, (1, 0, 2))

    mesh = plsc.VectorSubcoreMesh(core_axis_name="c", subcore_axis_name="s")
    f = pl.kernel(
        _sc_body,
        out_type=jax.ShapeDtypeStruct((NUM_SETS, b, D), jnp.float32),
        mesh=mesh,
        scratch_types=[
            pltpu.VMEM((NUM_SETS, SET_SIZE), jnp.float32),
            pltpu.VMEM((SET_SIZE, BCHUNK, D), jnp.float32),
            pltpu.VMEM((SET_SIZE, BCHUNK, D), jnp.float32),
            pltpu.VMEM((BCHUNK, D), jnp.float32),
            pltpu.SemaphoreType.DMA,
            pltpu.SemaphoreType.DMA,
        ],
    )
    out_sm = f(gf_t, w)
    return jnp.transpose(out_sm, (1, 0, 2))


# speedup vs baseline: 3.0703x; 1.1175x over previous
import functools

import jax
import jax.numpy as jnp
from jax import lax
from jax.experimental import pallas as pl
from jax.experimental.pallas import tpu as pltpu, tpu_sc as plsc

NUM_SETS = 32
SET_SIZE = 16
D = 128
NUM_GENES_USED = NUM_SETS * SET_SIZE
LANES = 16
DV = D // LANES

BCHUNK = 16


def _softmax_body(a_ref, o_ref):
    x = a_ref[...]
    m = jnp.max(x, axis=1, keepdims=True)
    e = jnp.exp(x - m)
    o_ref[...] = e / jnp.sum(e, axis=1, keepdims=True)


def _sc_body(gene_hbm, w_hbm, out_hbm, w_v, slab_a, slab_b, out_v, sem_a, sem_b):
    nc = 2
    wid = lax.axis_index("s") * nc + lax.axis_index("c")
    b = gene_hbm.shape[1]
    n_chunks = b // BCHUNK
    g0 = wid * SET_SIZE

    pltpu.sync_copy(w_hbm, w_v)
    wvec = w_v[wid, :]
    g_src = gene_hbm.at[pl.ds(g0, SET_SIZE)]

    def start_chunk(c, buf, sem):
        pltpu.make_async_copy(
            g_src.at[:, pl.ds(c * BCHUNK, BCHUNK), :], buf, sem
        ).start()

    def compute_chunk(buf, c):
        def b_body(bl, _):
            accs = [jnp.zeros((LANES,), jnp.float32) for _ in range(DV)]
            for k in range(SET_SIZE):
                wk = wvec[k]
                for v in range(DV):
                    accs[v] = accs[v] + wk * buf[k, bl, pl.ds(v * LANES, LANES)]
            for v in range(DV):
                out_v[bl, pl.ds(v * LANES, LANES)] = accs[v]
            return 0

        lax.fori_loop(0, BCHUNK, b_body, 0)
        pltpu.sync_copy(out_v, out_hbm.at[pl.ds(c * BCHUNK, BCHUNK), wid, :])

    start_chunk(0, slab_a, sem_a)

    def pair_body(i, _):
        c0 = 2 * i
        start_chunk(c0 + 1, slab_b, sem_b)
        pltpu.make_async_copy(
            g_src.at[:, pl.ds(c0 * BCHUNK, BCHUNK), :], slab_a, sem_a
        ).wait()
        compute_chunk(slab_a, c0)

        @pl.when(c0 + 2 < n_chunks)
        def _():
            start_chunk(c0 + 2, slab_a, sem_a)

        pltpu.make_async_copy(
            g_src.at[:, pl.ds((c0 + 1) * BCHUNK, BCHUNK), :], slab_b, sem_b
        ).wait()
        compute_chunk(slab_b, c0 + 1)
        return 0

    lax.fori_loop(0, n_chunks // 2, pair_body, 0)


def kernel(gene_features, attn_w):
    b = gene_features.shape[0]
    attn2 = attn_w.reshape(NUM_SETS, SET_SIZE)
    w = pl.pallas_call(
        _softmax_body,
        out_shape=jax.ShapeDtypeStruct((NUM_SETS, SET_SIZE), jnp.float32),
    )(attn2)

    gf_t = jnp.transpose(gene_features, (1, 0, 2))

    mesh = plsc.VectorSubcoreMesh(core_axis_name="c", subcore_axis_name="s")
    f = pl.kernel(
        _sc_body,
        out_type=jax.ShapeDtypeStruct((b, NUM_SETS, D), jnp.float32),
        mesh=mesh,
        scratch_types=[
            pltpu.VMEM((NUM_SETS, SET_SIZE), jnp.float32),
            pltpu.VMEM((SET_SIZE, BCHUNK, D), jnp.float32),
            pltpu.VMEM((SET_SIZE, BCHUNK, D), jnp.float32),
            pltpu.VMEM((BCHUNK, D), jnp.float32),
            pltpu.SemaphoreType.DMA,
            pltpu.SemaphoreType.DMA,
        ],
    )
    return f(gf_t, w)


# speedup vs baseline: 3.0840x; 1.0045x over previous
import functools

import jax
import jax.numpy as jnp
from jax import lax
from jax.experimental import pallas as pl
from jax.experimental.pallas import tpu as pltpu, tpu_sc as plsc

NUM_SETS = 32
SET_SIZE = 16
D = 128
NUM_GENES_USED = NUM_SETS * SET_SIZE
LANES = 16
DV = D // LANES

BCHUNK = 16


def _lane_shuffle(x, idx):
    return jnp.take_along_axis(x, idx, axis=0)


def _sc_body(gene_hbm, attn_hbm, out_hbm, attn_v, slab_a, slab_b, out_v, sem_a, sem_b):
    nc = 2
    wid = lax.axis_index("s") * nc + lax.axis_index("c")
    b = gene_hbm.shape[1]
    n_chunks = b // BCHUNK
    g0 = wid * SET_SIZE

    pltpu.sync_copy(attn_hbm, attn_v)
    av = attn_v[wid, :]
    lane = lax.iota(jnp.int32, LANES)
    m = av
    for sh in (8, 4, 2, 1):
        m = jnp.maximum(m, _lane_shuffle(m, lane ^ sh))
    e = jnp.exp(av - m)
    tot = e
    for sh in (8, 4, 2, 1):
        tot = tot + _lane_shuffle(tot, lane ^ sh)
    wvec = e / tot

    g_src = gene_hbm.at[pl.ds(g0, SET_SIZE)]

    def start_chunk(c, buf, sem):
        pltpu.make_async_copy(
            g_src.at[:, pl.ds(c * BCHUNK, BCHUNK), :], buf, sem
        ).start()

    def compute_chunk(buf, c):
        def b_body(bl, _):
            accs = [jnp.zeros((LANES,), jnp.float32) for _ in range(DV)]
            for k in range(SET_SIZE):
                wk = wvec[k]
                for v in range(DV):
                    accs[v] = accs[v] + wk * buf[k, bl, pl.ds(v * LANES, LANES)]
            for v in range(DV):
                out_v[bl, pl.ds(v * LANES, LANES)] = accs[v]
            return 0

        lax.fori_loop(0, BCHUNK, b_body, 0)
        pltpu.sync_copy(out_v, out_hbm.at[pl.ds(c * BCHUNK, BCHUNK), wid, :])

    start_chunk(0, slab_a, sem_a)

    def pair_body(i, _):
        c0 = 2 * i
        start_chunk(c0 + 1, slab_b, sem_b)
        pltpu.make_async_copy(
            g_src.at[:, pl.ds(c0 * BCHUNK, BCHUNK), :], slab_a, sem_a
        ).wait()
        compute_chunk(slab_a, c0)

        @pl.when(c0 + 2 < n_chunks)
        def _():
            start_chunk(c0 + 2, slab_a, sem_a)

        pltpu.make_async_copy(
            g_src.at[:, pl.ds((c0 + 1) * BCHUNK, BCHUNK), :], slab_b, sem_b
        ).wait()
        compute_chunk(slab_b, c0 + 1)
        return 0

    lax.fori_loop(0, n_chunks // 2, pair_body, 0)


def kernel(gene_features, attn_w):
    b = gene_features.shape[0]
    attn2 = attn_w.reshape(NUM_SETS, SET_SIZE)
    gf_t = jnp.transpose(gene_features, (1, 0, 2))

    mesh = plsc.VectorSubcoreMesh(core_axis_name="c", subcore_axis_name="s")
    f = pl.kernel(
        _sc_body,
        out_type=jax.ShapeDtypeStruct((b, NUM_SETS, D), jnp.float32),
        mesh=mesh,
        scratch_types=[
            pltpu.VMEM((NUM_SETS, SET_SIZE), jnp.float32),
            pltpu.VMEM((SET_SIZE, BCHUNK, D), jnp.float32),
            pltpu.VMEM((SET_SIZE, BCHUNK, D), jnp.float32),
            pltpu.VMEM((BCHUNK, D), jnp.float32),
            pltpu.SemaphoreType.DMA,
            pltpu.SemaphoreType.DMA,
        ],
    )
    return f(gf_t, attn2)


# speedup vs baseline: 3.0940x; 1.0032x over previous
import functools

import jax
import jax.numpy as jnp
from jax import lax
from jax.experimental import pallas as pl
from jax.experimental.pallas import tpu as pltpu, tpu_sc as plsc

NUM_SETS = 32
SET_SIZE = 16
D = 128
NUM_GENES_USED = NUM_SETS * SET_SIZE
LANES = 16
DV = D // LANES

BCHUNK = 8
SC_BATCH = 112
TC_TB = 16


def _lane_shuffle(x, idx):
    return jnp.take_along_axis(x, idx, axis=0)


def _sc_body(gene_hbm, attn_hbm, out_hbm, attn_v, slab_a, slab_b, out_v, sem_a, sem_b):
    nc = 2
    wid = lax.axis_index("s") * nc + lax.axis_index("c")
    n_chunks = out_hbm.shape[0] // BCHUNK
    g0 = wid * SET_SIZE

    pltpu.sync_copy(attn_hbm, attn_v)
    av = attn_v[wid, :]
    lane = lax.iota(jnp.int32, LANES)
    m = av
    for sh in (8, 4, 2, 1):
        m = jnp.maximum(m, _lane_shuffle(m, lane ^ sh))
    e = jnp.exp(av - m)
    tot = e
    for sh in (8, 4, 2, 1):
        tot = tot + _lane_shuffle(tot, lane ^ sh)
    wvec = e / tot

    g_src = gene_hbm.at[pl.ds(g0, SET_SIZE)]

    def start_chunk(c, buf, sem):
        pltpu.make_async_copy(
            g_src.at[:, pl.ds(c * BCHUNK, BCHUNK), :], buf, sem
        ).start()

    def compute_chunk(buf, c):
        def b_body(bl, _):
            accs = [jnp.zeros((LANES,), jnp.float32) for _ in range(DV)]
            for k in range(SET_SIZE):
                wk = wvec[k]
                for v in range(DV):
                    accs[v] = accs[v] + wk * buf[k, bl, pl.ds(v * LANES, LANES)]
            for v in range(DV):
                out_v[bl, pl.ds(v * LANES, LANES)] = accs[v]
            return 0

        lax.fori_loop(0, BCHUNK, b_body, 0)
        pltpu.sync_copy(out_v, out_hbm.at[pl.ds(c * BCHUNK, BCHUNK), wid, :])

    start_chunk(0, slab_a, sem_a)

    def pair_body(i, _):
        c0 = 2 * i
        start_chunk(c0 + 1, slab_b, sem_b)
        pltpu.make_async_copy(
            g_src.at[:, pl.ds(c0 * BCHUNK, BCHUNK), :], slab_a, sem_a
        ).wait()
        compute_chunk(slab_a, c0)

        @pl.when(c0 + 2 < n_chunks)
        def _():
            start_chunk(c0 + 2, slab_a, sem_a)

        pltpu.make_async_copy(
            g_src.at[:, pl.ds((c0 + 1) * BCHUNK, BCHUNK), :], slab_b, sem_b
        ).wait()
        compute_chunk(slab_b, c0 + 1)
        return 0

    lax.fori_loop(0, n_chunks // 2, pair_body, 0)


def _tc_pool_body(x_ref, a_ref, o_ref):
    a = a_ref[...]
    m = jnp.max(a, axis=1, keepdims=True)
    e = jnp.exp(a - m)
    w = e / jnp.sum(e, axis=1, keepdims=True)
    w_tiled = jnp.broadcast_to(w[:, None, :], (NUM_SETS, NUM_SETS, SET_SIZE))
    w_tiled = w_tiled.reshape(NUM_SETS, NUM_GENES_USED)
    gseg = jax.lax.broadcasted_iota(jnp.int32, (NUM_SETS, NUM_GENES_USED), 1) // SET_SIZE
    sidx = jax.lax.broadcasted_iota(jnp.int32, (NUM_SETS, NUM_GENES_USED), 0)
    wbd = jnp.where(gseg == sidx, w_tiled, 0.0)
    xr = x_ref[...].reshape(NUM_GENES_USED, TC_TB * D)
    y = jax.lax.dot_general(
        wbd, xr, (((1,), (0,)), ((), ())), preferred_element_type=jnp.float32
    )
    o_ref[...] = y.reshape(NUM_SETS, TC_TB, D)


def kernel(gene_features, attn_w):
    b = gene_features.shape[0]
    n_tc = b - SC_BATCH
    attn2 = attn_w.reshape(NUM_SETS, SET_SIZE)
    gf_t = jnp.transpose(gene_features, (1, 0, 2))

    mesh = plsc.VectorSubcoreMesh(core_axis_name="c", subcore_axis_name="s")
    f = pl.kernel(
        _sc_body,
        out_type=jax.ShapeDtypeStruct((SC_BATCH, NUM_SETS, D), jnp.float32),
        mesh=mesh,
        scratch_types=[
            pltpu.VMEM((NUM_SETS, SET_SIZE), jnp.float32),
            pltpu.VMEM((SET_SIZE, BCHUNK, D), jnp.float32),
            pltpu.VMEM((SET_SIZE, BCHUNK, D), jnp.float32),
            pltpu.VMEM((BCHUNK, D), jnp.float32),
            pltpu.SemaphoreType.DMA,
            pltpu.SemaphoreType.DMA,
        ],
    )
    out_sc = f(gf_t, attn2)

    out_tc = pl.pallas_call(
        _tc_pool_body,
        grid=(n_tc // TC_TB,),
        in_specs=[
            pl.BlockSpec(
                (NUM_GENES_USED, TC_TB, D),
                lambda j: (0, SC_BATCH // TC_TB + j, 0),
            ),
            pl.BlockSpec((NUM_SETS, SET_SIZE), lambda j: (0, 0)),
        ],
        out_specs=pl.BlockSpec((NUM_SETS, TC_TB, D), lambda j: (0, j, 0)),
        out_shape=jax.ShapeDtypeStruct((NUM_SETS, n_tc, D), jnp.float32),
    )(gf_t, attn2)

    return jnp.concatenate([out_sc, jnp.transpose(out_tc, (1, 0, 2))], axis=0)


# speedup vs baseline: 3.4047x; 1.1004x over previous
import functools

import jax
import jax.numpy as jnp
from jax import lax
from jax.experimental import pallas as pl
from jax.experimental.pallas import tpu as pltpu, tpu_sc as plsc

NUM_SETS = 32
SET_SIZE = 16
D = 128
NUM_GENES_USED = NUM_SETS * SET_SIZE
LANES = 16
DV = D // LANES

BCHUNK = 16
SC_BATCH = 96
TC_TB = 32


def _lane_shuffle(x, idx):
    return jnp.take_along_axis(x, idx, axis=0)


def _sc_body(gene_hbm, attn_hbm, out_hbm, attn_v, slab_a, slab_b, out_v, sem_a, sem_b):
    nc = 2
    wid = lax.axis_index("s") * nc + lax.axis_index("c")
    n_chunks = out_hbm.shape[0] // BCHUNK
    g0 = wid * SET_SIZE

    pltpu.sync_copy(attn_hbm, attn_v)
    av = attn_v[wid, :]
    lane = lax.iota(jnp.int32, LANES)
    m = av
    for sh in (8, 4, 2, 1):
        m = jnp.maximum(m, _lane_shuffle(m, lane ^ sh))
    e = jnp.exp(av - m)
    tot = e
    for sh in (8, 4, 2, 1):
        tot = tot + _lane_shuffle(tot, lane ^ sh)
    wvec = e / tot

    g_src = gene_hbm.at[pl.ds(g0, SET_SIZE)]

    def start_chunk(c, buf, sem):
        pltpu.make_async_copy(
            g_src.at[:, pl.ds(c * BCHUNK, BCHUNK), :], buf, sem
        ).start()

    def compute_chunk(buf, c):
        def b_body(bl, _):
            accs = [jnp.zeros((LANES,), jnp.float32) for _ in range(DV)]
            for k in range(SET_SIZE):
                wk = wvec[k]
                for v in range(DV):
                    accs[v] = accs[v] + wk * buf[k, bl, pl.ds(v * LANES, LANES)]
            for v in range(DV):
                out_v[bl, pl.ds(v * LANES, LANES)] = accs[v]
            return 0

        lax.fori_loop(0, BCHUNK, b_body, 0)
        pltpu.sync_copy(out_v, out_hbm.at[pl.ds(c * BCHUNK, BCHUNK), wid, :])

    start_chunk(0, slab_a, sem_a)

    def pair_body(i, _):
        c0 = 2 * i
        start_chunk(c0 + 1, slab_b, sem_b)
        pltpu.make_async_copy(
            g_src.at[:, pl.ds(c0 * BCHUNK, BCHUNK), :], slab_a, sem_a
        ).wait()
        compute_chunk(slab_a, c0)

        @pl.when(c0 + 2 < n_chunks)
        def _():
            start_chunk(c0 + 2, slab_a, sem_a)

        pltpu.make_async_copy(
            g_src.at[:, pl.ds((c0 + 1) * BCHUNK, BCHUNK), :], slab_b, sem_b
        ).wait()
        compute_chunk(slab_b, c0 + 1)
        return 0

    lax.fori_loop(0, n_chunks // 2, pair_body, 0)


def _tc_pool_body(x_ref, a_ref, o_ref):
    a = a_ref[...]
    m = jnp.max(a, axis=1, keepdims=True)
    e = jnp.exp(a - m)
    w = e / jnp.sum(e, axis=1, keepdims=True)
    w_tiled = jnp.broadcast_to(w[:, None, :], (NUM_SETS, NUM_SETS, SET_SIZE))
    w_tiled = w_tiled.reshape(NUM_SETS, NUM_GENES_USED)
    gseg = jax.lax.broadcasted_iota(jnp.int32, (NUM_SETS, NUM_GENES_USED), 1) // SET_SIZE
    sidx = jax.lax.broadcasted_iota(jnp.int32, (NUM_SETS, NUM_GENES_USED), 0)
    wbd = jnp.where(gseg == sidx, w_tiled, 0.0)
    xr = x_ref[...].reshape(NUM_GENES_USED, TC_TB * D)
    y = jax.lax.dot_general(
        wbd, xr, (((1,), (0,)), ((), ())), preferred_element_type=jnp.float32
    )
    o_ref[...] = jnp.swapaxes(y.reshape(NUM_SETS, TC_TB, D), 0, 1)


def kernel(gene_features, attn_w):
    b = gene_features.shape[0]
    n_tc = b - SC_BATCH
    attn2 = attn_w.reshape(NUM_SETS, SET_SIZE)
    gf_t = jnp.transpose(gene_features, (1, 0, 2))

    mesh = plsc.VectorSubcoreMesh(core_axis_name="c", subcore_axis_name="s")
    f = pl.kernel(
        _sc_body,
        out_type=jax.ShapeDtypeStruct((SC_BATCH, NUM_SETS, D), jnp.float32),
        mesh=mesh,
        scratch_types=[
            pltpu.VMEM((NUM_SETS, SET_SIZE), jnp.float32),
            pltpu.VMEM((SET_SIZE, BCHUNK, D), jnp.float32),
            pltpu.VMEM((SET_SIZE, BCHUNK, D), jnp.float32),
            pltpu.VMEM((BCHUNK, D), jnp.float32),
            pltpu.SemaphoreType.DMA,
            pltpu.SemaphoreType.DMA,
        ],
    )
    out_sc = f(gf_t, attn2)

    out_tc = pl.pallas_call(
        _tc_pool_body,
        grid=(n_tc // TC_TB,),
        in_specs=[
            pl.BlockSpec(
                (NUM_GENES_USED, TC_TB, D),
                lambda j: (0, SC_BATCH // TC_TB + j, 0),
            ),
            pl.BlockSpec((NUM_SETS, SET_SIZE), lambda j: (0, 0)),
        ],
        out_specs=pl.BlockSpec((TC_TB, NUM_SETS, D), lambda j: (j, 0, 0)),
        out_shape=jax.ShapeDtypeStruct((n_tc, NUM_SETS, D), jnp.float32),
    )(gf_t, attn2)

    return jnp.concatenate([out_sc, out_tc], axis=0)


# speedup vs baseline: 3.4979x; 1.0274x over previous
import functools

import jax
import jax.numpy as jnp
from jax import lax
from jax.experimental import pallas as pl
from jax.experimental.pallas import tpu as pltpu, tpu_sc as plsc

NUM_SETS = 32
SET_SIZE = 16
D = 128
NUM_GENES_USED = NUM_SETS * SET_SIZE
LANES = 16
DV = D // LANES

BCHUNK = 16
SC_BATCH = 80
TC_TB = 16


def _lane_shuffle(x, idx):
    return jnp.take_along_axis(x, idx, axis=0)


def _sc_body(gene_hbm, attn_hbm, out_hbm, attn_v, slab_a, slab_b, out_v, sem_a, sem_b):
    nc = 2
    wid = lax.axis_index("s") * nc + lax.axis_index("c")
    n_chunks = out_hbm.shape[0] // BCHUNK
    g0 = wid * SET_SIZE

    pltpu.sync_copy(attn_hbm, attn_v)
    av = attn_v[wid, :]
    lane = lax.iota(jnp.int32, LANES)
    m = av
    for sh in (8, 4, 2, 1):
        m = jnp.maximum(m, _lane_shuffle(m, lane ^ sh))
    e = jnp.exp(av - m)
    tot = e
    for sh in (8, 4, 2, 1):
        tot = tot + _lane_shuffle(tot, lane ^ sh)
    wvec = e / tot

    g_src = gene_hbm.at[pl.ds(g0, SET_SIZE)]

    def start_chunk(c, buf, sem):
        pltpu.make_async_copy(
            g_src.at[:, pl.ds(c * BCHUNK, BCHUNK), :], buf, sem
        ).start()

    def compute_chunk(buf, c):
        def b_body(bl, _):
            accs = [jnp.zeros((LANES,), jnp.float32) for _ in range(DV)]
            for k in range(SET_SIZE):
                wk = wvec[k]
                for v in range(DV):
                    accs[v] = accs[v] + wk * buf[k, bl, pl.ds(v * LANES, LANES)]
            for v in range(DV):
                out_v[bl, pl.ds(v * LANES, LANES)] = accs[v]
            return 0

        lax.fori_loop(0, BCHUNK, b_body, 0)
        pltpu.sync_copy(out_v, out_hbm.at[pl.ds(c * BCHUNK, BCHUNK), wid, :])

    start_chunk(0, slab_a, sem_a)

    def pair_body(i, _):
        c0 = 2 * i
        start_chunk(c0 + 1, slab_b, sem_b)
        pltpu.make_async_copy(
            g_src.at[:, pl.ds(c0 * BCHUNK, BCHUNK), :], slab_a, sem_a
        ).wait()
        compute_chunk(slab_a, c0)

        @pl.when(c0 + 2 < n_chunks)
        def _():
            start_chunk(c0 + 2, slab_a, sem_a)

        pltpu.make_async_copy(
            g_src.at[:, pl.ds((c0 + 1) * BCHUNK, BCHUNK), :], slab_b, sem_b
        ).wait()
        compute_chunk(slab_b, c0 + 1)
        return 0

    lax.fori_loop(0, n_chunks // 2, pair_body, 0)

    if n_chunks % 2:
        c_last = n_chunks - 1
        pltpu.make_async_copy(
            g_src.at[:, pl.ds(c_last * BCHUNK, BCHUNK), :], slab_a, sem_a
        ).wait()
        compute_chunk(slab_a, c_last)


def _tc_pool_body(x_ref, a_ref, o_ref):
    a = a_ref[...]
    m = jnp.max(a, axis=1, keepdims=True)
    e = jnp.exp(a - m)
    w = e / jnp.sum(e, axis=1, keepdims=True)
    x4 = x_ref[...].reshape(NUM_SETS, SET_SIZE, TC_TB, D)
    y = jnp.sum(x4 * w[:, :, None, None], axis=1)
    o_ref[...] = jnp.swapaxes(y, 0, 1)


def kernel(gene_features, attn_w):
    b = gene_features.shape[0]
    n_tc = b - SC_BATCH
    attn2 = attn_w.reshape(NUM_SETS, SET_SIZE)
    gf_t = jnp.transpose(gene_features, (1, 0, 2))

    mesh = plsc.VectorSubcoreMesh(core_axis_name="c", subcore_axis_name="s")
    f = pl.kernel(
        _sc_body,
        out_type=jax.ShapeDtypeStruct((SC_BATCH, NUM_SETS, D), jnp.float32),
        mesh=mesh,
        scratch_types=[
            pltpu.VMEM((NUM_SETS, SET_SIZE), jnp.float32),
            pltpu.VMEM((SET_SIZE, BCHUNK, D), jnp.float32),
            pltpu.VMEM((SET_SIZE, BCHUNK, D), jnp.float32),
            pltpu.VMEM((BCHUNK, D), jnp.float32),
            pltpu.SemaphoreType.DMA,
            pltpu.SemaphoreType.DMA,
        ],
    )
    out_sc = f(gf_t, attn2)

    out_tc = pl.pallas_call(
        _tc_pool_body,
        grid=(n_tc // TC_TB,),
        in_specs=[
            pl.BlockSpec(
                (NUM_GENES_USED, TC_TB, D),
                lambda j: (0, SC_BATCH // TC_TB + j, 0),
            ),
            pl.BlockSpec((NUM_SETS, SET_SIZE), lambda j: (0, 0)),
        ],
        out_specs=pl.BlockSpec((TC_TB, NUM_SETS, D), lambda j: (j, 0, 0)),
        out_shape=jax.ShapeDtypeStruct((n_tc, NUM_SETS, D), jnp.float32),
    )(gf_t, attn2)

    return jnp.concatenate([out_sc, out_tc], axis=0)
